# baseline (speedup 1.0000x reference)
import jax
import jax.numpy as jnp
import numpy as np
from jax.experimental import pallas as pl
from jax.experimental.pallas import tpu as pltpu

B, T, D, K_REP, CS, H = 4, 2048, 1024, 16, 48, 8
DH = D // H
TB = 256

_MIN32 = np.int32(-2147483648)
_M31 = np.int32(0x7FFFFFFF)


def _dotT(x, w):
    return jax.lax.dot_general(x, w, (((1,), (1,)), ((), ())),
                               preferred_element_type=jnp.float32)


def _mlp(q, W1, b1, g, be, W2, b2):
    h = _dotT(q, W1) + b1
    m = h.mean(-1, keepdims=True)
    v = ((h - m) ** 2).mean(-1, keepdims=True)
    h = (h - m) / jnp.sqrt(v + 1e-5) * g + be
    h = jnp.maximum(h, 0.0)
    return _dotT(h, W2) + b2


def _prep_body(q_ref, mw1, mb1, mg, mbe, mw2, mb2,
               sw1, sb1, sg_, sbe, sw2, sb2, mu_ref, sig_ref):
    q = q_ref[...]
    mu_ref[...] = _mlp(q, mw1[...], mb1[...], mg[...], mbe[...], mw2[...], mb2[...])
    s = _mlp(q, sw1[...], sb1[...], sg_[...], sbe[...], sw2[...], sb2[...])
    sig_ref[...] = jnp.maximum(s, 0.0) + jnp.log1p(jnp.exp(-jnp.abs(s))) + 1e-6


def _dist_body(v_ref, mu_ref, sig_ref, d_ref):
    v = v_ref[...]
    c = v - mu_ref[...][:, None, :]
    rs = 1.0 / sig_ref[...]
    d_ref[...] = jnp.sum(c * c * rs[:, None, :], axis=-1)


def _select_body(d_ref, idx_ref, rows_ref):
    d = d_ref[...]
    bits = jax.lax.bitcast_convert_type(d, jnp.int32)
    u = bits ^ (jax.lax.shift_right_arithmetic(bits, 31) & _M31)
    u = u ^ _MIN32
    rank = jnp.full((B, 1), (T - 1) // 2, jnp.int32)
    cand = jnp.full((B, T), True)
    for i in range(31, -1, -1):
        bit = jax.lax.shift_right_logical(u, i) & 1
        zero = bit == 0
        c0 = jnp.sum(jnp.where(cand & zero, 1, 0), axis=1, keepdims=True)
        go0 = rank < c0
        rank = jnp.where(go0, rank, rank - c0)
        cand = cand & (zero == go0)
    target = jnp.min(jnp.where(cand, d, jnp.inf), axis=1, keepdims=True)

    d2t = jnp.abs(d - target)
    iota_t = jax.lax.broadcasted_iota(jnp.int32, (B, T), 1)
    work = d2t
    cols = []
    for _ in range(CS):
        m = jnp.min(work, axis=1, keepdims=True)
        pos = jnp.min(jnp.where(work == m, iota_t, T), axis=1, keepdims=True)
        cols.append(pos)
        work = jnp.where(iota_t == pos, jnp.inf, work)
    cand_arr = jnp.concatenate(cols, axis=1)

    iota_c = jax.lax.broadcasted_iota(jnp.int32, (B, CS), 1)
    first = cand_arr[:, 0:1]
    removed = iota_c == 0
    md = jnp.abs(cand_arr - first)
    sel = [first]
    for _ in range(K_REP - 1):
        scores = jnp.where(removed, -1, md)
        mx = jnp.max(scores, axis=1, keepdims=True)
        pos = jnp.min(jnp.where(scores == mx, iota_c, CS), axis=1, keepdims=True)
        hit = iota_c == pos
        new = jnp.sum(jnp.where(hit, cand_arr, 0), axis=1, keepdims=True)
        removed = removed | hit
        md = jnp.minimum(md, jnp.abs(cand_arr - new))
        sel.append(new)
    idx = jnp.concatenate(sel, axis=1)
    idx_ref[...] = idx
    rows_ref[...] = idx + jax.lax.broadcasted_iota(jnp.int32, (B, K_REP), 0) * T


def _gather_body(rows_ref, v_ref, rep_ref):
    rep_ref[...] = v_ref[...]


def _mha_body(rep_ref, win, bin_, wout, bout, out_ref, aw_ref):
    x = rep_ref[...].reshape(B * K_REP, D)
    qkv = _dotT(x, win[...]) + bin_[...]
    scale = np.float32(1.0 / np.sqrt(DH))
    outs, aws = [], []
    for b in range(B):
        qkvb = qkv[b * K_REP:(b + 1) * K_REP]
        acc = jnp.zeros((K_REP, K_REP), jnp.float32)
        heads = []
        for h in range(H):
            qh = qkvb[:, h * DH:(h + 1) * DH]
            kh = qkvb[:, D + h * DH:D + (h + 1) * DH]
            vh = qkvb[:, 2 * D + h * DH:2 * D + (h + 1) * DH]
            s = jax.lax.dot_general(qh, kh, (((1,), (1,)), ((), ())),
                                    preferred_element_type=jnp.float32) * scale
            s = s - jnp.max(s, axis=-1, keepdims=True)
            e = jnp.exp(s)
            a = e / jnp.sum(e, axis=-1, keepdims=True)
            acc = acc + a
            heads.append(jnp.dot(a, vh, preferred_element_type=jnp.float32))
        o = jnp.concatenate(heads, axis=1)
        outs.append(_dotT(o, wout[...]) + bout[...])
        aws.append(acc * np.float32(1.0 / H))
    out_ref[...] = jnp.stack(outs, axis=0)
    aw_ref[...] = jnp.stack(aws, axis=0)


def kernel(video_features, query_embedding, mu_W1, mu_b1, mu_g, mu_be, mu_W2,
           mu_b2, sg_W1, sg_b1, sg_g, sg_be, sg_W2, sg_b2, attn_in_W,
           attn_in_b, attn_out_W, attn_out_b):
    f32 = jnp.float32
    mu_q, sigma = pl.pallas_call(
        _prep_body,
        out_shape=(jax.ShapeDtypeStruct((B, D), f32),
                   jax.ShapeDtypeStruct((B, D), f32)),
    )(query_embedding, mu_W1, mu_b1, mu_g, mu_be, mu_W2, mu_b2,
      sg_W1, sg_b1, sg_g, sg_be, sg_W2, sg_b2)

    dist = pl.pallas_call(
        _dist_body,
        grid=(T // TB,),
        in_specs=[
            pl.BlockSpec((B, TB, D), lambda i: (0, i, 0)),
            pl.BlockSpec((B, D), lambda i: (0, 0)),
            pl.BlockSpec((B, D), lambda i: (0, 0)),
        ],
        out_specs=pl.BlockSpec((B, TB), lambda i: (0, i)),
        out_shape=jax.ShapeDtypeStruct((B, T), f32),
    )(video_features, mu_q, sigma)

    idx, rows = pl.pallas_call(
        _select_body,
        out_shape=(jax.ShapeDtypeStruct((B, K_REP), jnp.int32),
                   jax.ShapeDtypeStruct((B, K_REP), jnp.int32)),
    )(dist)

    video_r = video_features.reshape(B * T, 1, D)
    rep_r = pl.pallas_call(
        _gather_body,
        grid_spec=pltpu.PrefetchScalarGridSpec(
            num_scalar_prefetch=1,
            grid=(B, K_REP),
            in_specs=[pl.BlockSpec((1, 1, D),
                                   lambda b, k, rref: (rref[b, k], 0, 0))],
            out_specs=pl.BlockSpec((1, 1, D),
                                   lambda b, k, rref: (b * K_REP + k, 0, 0)),
        ),
        out_shape=jax.ShapeDtypeStruct((B * K_REP, 1, D), f32),
    )(rows, video_r)
    rep = rep_r.reshape(B, K_REP, D)

    refined, attn_w = pl.pallas_call(
        _mha_body,
        out_shape=(jax.ShapeDtypeStruct((B, K_REP, D), f32),
                   jax.ShapeDtypeStruct((B, K_REP, K_REP), f32)),
    )(rep, attn_in_W, attn_in_b, attn_out_W, attn_out_b)

    return refined, idx, dist, mu_q, sigma, attn_w
